# Initial kernel scaffold; baseline (speedup 1.0000x reference)
import jax
import jax.numpy as jnp
from jax.experimental import pallas as pl


def kernel(weak_feat, hard_feat, lb_feat, lb_one_hot, logits_x_lb, logits_x_ulb_1, logits_x_ulb_2, y_lb, y_ulb):
    raise NotImplementedError("write your pallas kernel here")


# speedup vs baseline: 1.7125x; 1.7125x over previous
import functools

import jax
import jax.numpy as jnp
from jax import lax
from jax.experimental import pallas as pl
from jax.experimental.pallas import tpu as pltpu
from jax.experimental.pallas import tpu_sc as plsc

NUM_CLASSES = 10
FEAT_DIM = 128
TAU = 0.5
P_CUTOFF = 0.5
B_ULB = 16384
B_LB = 4096
NC, NS = 2, 16
NW = NC * NS
ACC_ROWS = 16
CHUNK = 128
ULB_PER_W = B_ULB // NW
LB_PER_W = B_LB // NW


def _prep_body(logits_ref, onehot_ref, idx_ref, cnt_ref):
    l = logits_ref[...]
    m = jnp.max(l, axis=1, keepdims=True)
    s = jnp.sum(jnp.exp(l - m), axis=1, keepdims=True)
    mask = (1.0 / s) > P_CUTOFF
    col = lax.broadcasted_iota(jnp.int32, l.shape, 1)
    is_max = l == m
    amax = jnp.min(jnp.where(is_max, col, NUM_CLASSES), axis=1, keepdims=True)
    idx_ref[...] = jnp.where(mask, amax, NUM_CLASSES)
    oh = jnp.where((col == amax) & mask, 1.0, 0.0)
    cnt_ulb = jnp.sum(oh, axis=0)
    cnt_lb = jnp.sum(onehot_ref[...], axis=0)
    cnt_ref[0, :] = cnt_lb + cnt_ulb


def _sc_scatter_body(weak_hbm, idxu_hbm, lb_hbm, idxl_hbm, out_hbm,
                     buf, idxv, zbuf, shared):
    cid = lax.axis_index("c")
    sid = lax.axis_index("s")
    wid = cid * NS + sid
    zero = jnp.zeros((16,), jnp.float32)
    for i in range(ACC_ROWS):
        for j in range(FEAT_DIM // 16):
            zbuf[i, pl.ds(j * 16, 16)] = zero

    @pl.when(sid == 0)
    def _zero_acc():
        pltpu.sync_copy(zbuf, shared)

    plsc.subcore_barrier()

    pltpu.sync_copy(idxl_hbm.at[pl.ds(wid * LB_PER_W, CHUNK)], idxv)
    pltpu.sync_copy(lb_hbm.at[pl.ds(wid * LB_PER_W, CHUNK)], buf)
    pltpu.sync_copy(buf, shared.at[idxv], add=True)

    for j in range(ULB_PER_W // CHUNK):
        base = wid * ULB_PER_W + j * CHUNK
        pltpu.sync_copy(idxu_hbm.at[pl.ds(base, CHUNK)], idxv)
        pltpu.sync_copy(weak_hbm.at[pl.ds(base, CHUNK)], buf)
        pltpu.sync_copy(buf, shared.at[idxv], add=True)

    plsc.subcore_barrier()

    @pl.when(sid == 0)
    def _writeback():
        pltpu.sync_copy(shared, zbuf)
        pltpu.sync_copy(zbuf, out_hbm.at[cid])


def _atten_body(w_ref, part_ref, cnt_ref, out_ref):
    psum = part_ref[0, :NUM_CLASSES, :] + part_ref[1, :NUM_CLASSES, :]
    cnt = cnt_ref[0, :]
    p = psum / cnt[:, None]
    p = p / jnp.maximum(
        jnp.sqrt(jnp.sum(p * p, axis=1, keepdims=True)), 1e-12)
    w = w_ref[...]
    wn = w / jnp.maximum(
        jnp.sqrt(jnp.sum(w * w, axis=1, keepdims=True)), 1e-12)
    logits = lax.dot_general(
        wn, p, (((1,), (1,)), ((), ())),
        preferred_element_type=jnp.float32) * (1.0 / TAU)
    m = jnp.max(logits, axis=1, keepdims=True)
    e = jnp.exp(logits - m)
    out_ref[...] = e / jnp.sum(e, axis=1, keepdims=True)


def kernel(weak_feat, hard_feat, lb_feat, lb_one_hot, logits_x_lb,
           logits_x_ulb_1, logits_x_ulb_2, y_lb, y_ulb):
    idx_ulb, class_num = pl.pallas_call(
        _prep_body,
        out_shape=[
            jax.ShapeDtypeStruct((B_ULB, 1), jnp.int32),
            jax.ShapeDtypeStruct((1, NUM_CLASSES), jnp.float32),
        ],
    )(logits_x_ulb_1, lb_one_hot)

    idx_ulb = idx_ulb.reshape(B_ULB)
    idx_lb = y_lb.astype(jnp.int32).reshape(B_LB)

    mesh = plsc.VectorSubcoreMesh(
        core_axis_name="c", subcore_axis_name="s",
        num_cores=NC, num_subcores=NS)
    sc_scatter = functools.partial(
        pl.kernel,
        out_type=jax.ShapeDtypeStruct((NC, ACC_ROWS, FEAT_DIM), jnp.float32),
        mesh=mesh,
        scratch_types=[
            pltpu.VMEM((CHUNK, FEAT_DIM), jnp.float32),
            pltpu.VMEM((CHUNK,), jnp.int32),
            pltpu.VMEM((ACC_ROWS, FEAT_DIM), jnp.float32),
            pltpu.VMEM_SHARED((ACC_ROWS, FEAT_DIM), jnp.float32),
        ],
    )(_sc_scatter_body)
    partials = sc_scatter(weak_feat, idx_ulb, lb_feat, idx_lb)

    n_blocks = 16
    blk = B_ULB // n_blocks
    agg_out = pl.pallas_call(
        _atten_body,
        grid=(n_blocks,),
        in_specs=[
            pl.BlockSpec((blk, FEAT_DIM), lambda i: (i, 0)),
            pl.BlockSpec((NC, ACC_ROWS, FEAT_DIM), lambda i: (0, 0, 0)),
            pl.BlockSpec((1, NUM_CLASSES), lambda i: (0, 0)),
        ],
        out_specs=pl.BlockSpec((blk, NUM_CLASSES), lambda i: (i, 0)),
        out_shape=jax.ShapeDtypeStruct((B_ULB, NUM_CLASSES), jnp.float32),
    )(weak_feat, partials, class_num)

    return (weak_feat, hard_feat, lb_feat, lb_one_hot, logits_x_lb,
            agg_out, logits_x_ulb_2)


# speedup vs baseline: 2.0708x; 1.2092x over previous
import functools

import jax
import jax.numpy as jnp
from jax import lax
from jax.experimental import pallas as pl
from jax.experimental.pallas import tpu as pltpu
from jax.experimental.pallas import tpu_sc as plsc

NUM_CLASSES = 10
FEAT_DIM = 128
TAU = 0.5
P_CUTOFF = 0.5
B_ULB = 16384
B_LB = 4096
NC, NS = 2, 16
NW = NC * NS
ACC_ROWS = 16
CHUNK = 128
ULB_PER_W = B_ULB // NW
LB_PER_W = B_LB // NW
N_CHUNKS = ULB_PER_W // CHUNK + 1


def _prep_body(lT_ref, ohT_ref, idx_ref, cnt_ref):
    lT = lT_ref[...]
    m = jnp.max(lT, axis=0, keepdims=True)
    e = jnp.exp(lT - m)
    s = jnp.sum(e, axis=0, keepdims=True)
    mask = (1.0 / s) > P_CUTOFF
    row = lax.broadcasted_iota(jnp.int32, lT.shape, 0)
    is_max = lT == m
    amax = jnp.min(jnp.where(is_max, row, NUM_CLASSES), axis=0, keepdims=True)
    idx_ref[...] = jnp.where(mask, amax, NUM_CLASSES)
    oh = jnp.where((row == amax) & mask, 1.0, 0.0)
    cnt_ref[...] = (jnp.sum(oh, axis=1, keepdims=True)
                    + jnp.sum(ohT_ref[...], axis=1, keepdims=True))


def _sc_scatter_body(weak_hbm, idxu_hbm, lb_hbm, idxl_hbm, out_hbm,
                     buf, idxv, zbuf, sems, shared):
    cid = lax.axis_index("c")
    sid = lax.axis_index("s")
    wid = cid * NS + sid
    zero = jnp.zeros((16,), jnp.float32)
    for i in range(ACC_ROWS):
        for j in range(FEAT_DIM // 16):
            zbuf[i, pl.ds(j * 16, 16)] = zero

    @pl.when(sid == 0)
    def _zero_acc():
        pltpu.sync_copy(zbuf, shared)

    plsc.subcore_barrier()

    def src(k):
        if k < N_CHUNKS - 1:
            base = wid * ULB_PER_W + k * CHUNK
            return weak_hbm.at[pl.ds(base, CHUNK)], idxu_hbm.at[pl.ds(base, CHUNK)]
        base = wid * LB_PER_W
        return lb_hbm.at[pl.ds(base, CHUNK)], idxl_hbm.at[pl.ds(base, CHUNK)]

    f0, i0 = src(0)
    pltpu.async_copy(f0, buf.at[0], sems.at[0]).wait()
    pltpu.async_copy(i0, idxv.at[0], sems.at[0]).wait()
    for k in range(N_CHUNKS):
        b = k % 2
        if k + 1 < N_CHUNKS:
            nb = (k + 1) % 2
            fk, ik = src(k + 1)
            cf = pltpu.async_copy(fk, buf.at[nb], sems.at[nb])
            ci = pltpu.async_copy(ik, idxv.at[nb], sems.at[nb])
        pltpu.sync_copy(buf.at[b], shared.at[idxv.at[b]], add=True)
        if k + 1 < N_CHUNKS:
            cf.wait()
            ci.wait()

    plsc.subcore_barrier()

    @pl.when(sid == 0)
    def _writeback():
        pltpu.sync_copy(shared, zbuf)
        pltpu.sync_copy(zbuf, out_hbm.at[cid])


def _sc_echo_body(weak_hbm, hard_hbm, lb_hbm, oh_hbm, llb_hbm, l2_hbm,
                  weak_o, hard_o, lb_o, oh_o, llb_o, l2_o,
                  buf, sbuf, sems):
    cid = lax.axis_index("c")
    sid = lax.axis_index("s")
    wid = cid * NS + sid

    jobs = []
    for j in range(4):
        r = wid * ULB_PER_W + j * CHUNK
        jobs.append((weak_hbm.at[pl.ds(r, CHUNK)], weak_o.at[pl.ds(r, CHUNK)]))
        jobs.append((hard_hbm.at[pl.ds(r, CHUNK)], hard_o.at[pl.ds(r, CHUNK)]))
    r = wid * LB_PER_W
    jobs.append((lb_hbm.at[pl.ds(r, CHUNK)], lb_o.at[pl.ds(r, CHUNK)]))

    s0, d0 = jobs[0]
    pltpu.async_copy(s0, buf.at[0], sems.at[0]).wait()
    for k in range(len(jobs)):
        b = k % 2
        if k + 1 < len(jobs):
            nb = (k + 1) % 2
            cr = pltpu.async_copy(jobs[k + 1][0], buf.at[nb], sems.at[nb])
        cw = pltpu.async_copy(buf.at[b], jobs[k][1], sems.at[2])
        cw.wait()
        if k + 1 < len(jobs):
            cr.wait()

    ru = wid * ULB_PER_W
    pltpu.sync_copy(l2_hbm.at[pl.ds(ru, ULB_PER_W)], sbuf)
    pltpu.sync_copy(sbuf, l2_o.at[pl.ds(ru, ULB_PER_W)])
    rl = wid * LB_PER_W
    pltpu.sync_copy(oh_hbm.at[pl.ds(rl, LB_PER_W)], sbuf.at[pl.ds(0, LB_PER_W)])
    pltpu.sync_copy(sbuf.at[pl.ds(0, LB_PER_W)], oh_o.at[pl.ds(rl, LB_PER_W)])
    pltpu.sync_copy(llb_hbm.at[pl.ds(rl, LB_PER_W)], sbuf.at[pl.ds(0, LB_PER_W)])
    pltpu.sync_copy(sbuf.at[pl.ds(0, LB_PER_W)], llb_o.at[pl.ds(rl, LB_PER_W)])


def _atten_body(w_ref, part_ref, cnt_ref, out_ref):
    psum = part_ref[0, :NUM_CLASSES, :] + part_ref[1, :NUM_CLASSES, :]
    p = psum / cnt_ref[...]
    p = p / jnp.maximum(
        jnp.sqrt(jnp.sum(p * p, axis=1, keepdims=True)), 1e-12)
    w = w_ref[...]
    wn = w / jnp.maximum(
        jnp.sqrt(jnp.sum(w * w, axis=1, keepdims=True)), 1e-12)
    logits = lax.dot_general(
        wn, p, (((1,), (1,)), ((), ())),
        preferred_element_type=jnp.float32) * (1.0 / TAU)
    m = jnp.max(logits, axis=1, keepdims=True)
    e = jnp.exp(logits - m)
    out_ref[...] = e / jnp.sum(e, axis=1, keepdims=True)


def kernel(weak_feat, hard_feat, lb_feat, lb_one_hot, logits_x_lb,
           logits_x_ulb_1, logits_x_ulb_2, y_lb, y_ulb):
    idx_row, class_num = pl.pallas_call(
        _prep_body,
        out_shape=[
            jax.ShapeDtypeStruct((1, B_ULB), jnp.int32),
            jax.ShapeDtypeStruct((NUM_CLASSES, 1), jnp.float32),
        ],
    )(logits_x_ulb_1.T, lb_one_hot.T)

    idx_ulb = idx_row.reshape(B_ULB)
    idx_lb = y_lb.astype(jnp.int32).reshape(B_LB)

    mesh = plsc.VectorSubcoreMesh(
        core_axis_name="c", subcore_axis_name="s",
        num_cores=NC, num_subcores=NS)

    sc_scatter = functools.partial(
        pl.kernel,
        out_type=jax.ShapeDtypeStruct((NC, ACC_ROWS, FEAT_DIM), jnp.float32),
        mesh=mesh,
        scratch_types=[
            pltpu.VMEM((2, CHUNK, FEAT_DIM), jnp.float32),
            pltpu.VMEM((2, CHUNK), jnp.int32),
            pltpu.VMEM((ACC_ROWS, FEAT_DIM), jnp.float32),
            pltpu.SemaphoreType.DMA((2,)),
            pltpu.VMEM_SHARED((ACC_ROWS, FEAT_DIM), jnp.float32),
        ],
    )(_sc_scatter_body)
    partials = sc_scatter(weak_feat, idx_ulb, lb_feat, idx_lb)

    (partials, weak_e, hard_e, lb_e, oh_e, llb_e, l2_e) = \
        lax.optimization_barrier(
            (partials, weak_feat, hard_feat, lb_feat, lb_one_hot,
             logits_x_lb, logits_x_ulb_2))

    sc_echo = functools.partial(
        pl.kernel,
        out_type=(
            jax.ShapeDtypeStruct((B_ULB, FEAT_DIM), jnp.float32),
            jax.ShapeDtypeStruct((B_ULB, FEAT_DIM), jnp.float32),
            jax.ShapeDtypeStruct((B_LB, FEAT_DIM), jnp.float32),
            jax.ShapeDtypeStruct((B_LB, NUM_CLASSES), jnp.float32),
            jax.ShapeDtypeStruct((B_LB, NUM_CLASSES), jnp.float32),
            jax.ShapeDtypeStruct((B_ULB, NUM_CLASSES), jnp.float32),
        ),
        mesh=mesh,
        scratch_types=[
            pltpu.VMEM((2, CHUNK, FEAT_DIM), jnp.float32),
            pltpu.VMEM((ULB_PER_W, NUM_CLASSES), jnp.float32),
            pltpu.SemaphoreType.DMA((3,)),
        ],
    )(_sc_echo_body)
    weak_o, hard_o, lb_o, oh_o, llb_o, l2_o = sc_echo(
        weak_e, hard_e, lb_e, oh_e, llb_e, l2_e)

    n_blocks = 16
    blk = B_ULB // n_blocks
    agg_out = pl.pallas_call(
        _atten_body,
        grid=(n_blocks,),
        in_specs=[
            pl.BlockSpec((blk, FEAT_DIM), lambda i: (i, 0)),
            pl.BlockSpec((NC, ACC_ROWS, FEAT_DIM), lambda i: (0, 0, 0)),
            pl.BlockSpec((NUM_CLASSES, 1), lambda i: (0, 0)),
        ],
        out_specs=pl.BlockSpec((blk, NUM_CLASSES), lambda i: (i, 0)),
        out_shape=jax.ShapeDtypeStruct((B_ULB, NUM_CLASSES), jnp.float32),
    )(weak_feat, partials, class_num)

    return (weak_o, hard_o, lb_o, oh_o, llb_o, agg_out, l2_o)


# speedup vs baseline: 3.0698x; 1.4824x over previous
import functools

import jax
import jax.numpy as jnp
from jax import lax
from jax.experimental import pallas as pl
from jax.experimental.pallas import tpu as pltpu
from jax.experimental.pallas import tpu_sc as plsc

NUM_CLASSES = 10
FEAT_DIM = 128
TAU = 0.5
P_CUTOFF = 0.5
B_ULB = 16384
B_LB = 4096
NC, NS = 2, 16
NW = NC * NS
ACC_ROWS = 16
CHUNK = 128
ULB_PER_W = B_ULB // NW
LB_PER_W = B_LB // NW
N_CHUNKS = ULB_PER_W // CHUNK + 1


def _prep_body(lT_ref, ohT_ref, idx_ref, cnt_ref):
    lT = lT_ref[...]
    m = jnp.max(lT, axis=0, keepdims=True)
    e = jnp.exp(lT - m)
    s = jnp.sum(e, axis=0, keepdims=True)
    mask = (1.0 / s) > P_CUTOFF
    row = lax.broadcasted_iota(jnp.int32, lT.shape, 0)
    is_max = lT == m
    amax = jnp.min(jnp.where(is_max, row, NUM_CLASSES), axis=0, keepdims=True)
    idx_ref[...] = jnp.where(mask, amax, NUM_CLASSES)
    oh = jnp.where((row == amax) & mask, 1.0, 0.0)
    cnt_ref[...] = (jnp.sum(oh, axis=1, keepdims=True)
                    + jnp.sum(ohT_ref[...], axis=1, keepdims=True))


def _sc_scatter_body(weak_hbm, idxu_hbm, lb_hbm, idxl_hbm, out_hbm,
                     buf, idxv, zbuf, sems, shared):
    cid = lax.axis_index("c")
    sid = lax.axis_index("s")
    wid = cid * NS + sid
    zero = jnp.zeros((16,), jnp.float32)
    for i in range(ACC_ROWS):
        for j in range(FEAT_DIM // 16):
            zbuf[i, pl.ds(j * 16, 16)] = zero

    @pl.when(sid == 0)
    def _zero_acc():
        pltpu.sync_copy(zbuf, shared)

    plsc.subcore_barrier()

    def src(k):
        if k < N_CHUNKS - 1:
            base = wid * ULB_PER_W + k * CHUNK
            return weak_hbm.at[pl.ds(base, CHUNK)], idxu_hbm.at[pl.ds(base, CHUNK)]
        base = wid * LB_PER_W
        return lb_hbm.at[pl.ds(base, CHUNK)], idxl_hbm.at[pl.ds(base, CHUNK)]

    f0, i0 = src(0)
    pltpu.async_copy(f0, buf.at[0], sems.at[0]).wait()
    pltpu.async_copy(i0, idxv.at[0], sems.at[0]).wait()
    for k in range(N_CHUNKS):
        b = k % 2
        if k + 1 < N_CHUNKS:
            nb = (k + 1) % 2
            fk, ik = src(k + 1)
            cf = pltpu.async_copy(fk, buf.at[nb], sems.at[nb])
            ci = pltpu.async_copy(ik, idxv.at[nb], sems.at[nb])
        pltpu.sync_copy(buf.at[b], shared.at[idxv.at[b]], add=True)
        if k + 1 < N_CHUNKS:
            cf.wait()
            ci.wait()

    plsc.subcore_barrier()

    @pl.when(sid == 0)
    def _writeback():
        pltpu.sync_copy(shared, zbuf)
        pltpu.sync_copy(zbuf, out_hbm.at[cid])


def _sc_echo_body(weak_hbm, hard_hbm, lb_hbm,
                  weak_o, hard_o, lb_o,
                  buf, sems):
    cid = lax.axis_index("c")
    sid = lax.axis_index("s")
    wid = cid * NS + sid

    jobs = []
    for j in range(4):
        r = wid * ULB_PER_W + j * CHUNK
        jobs.append((weak_hbm.at[pl.ds(r, CHUNK)], weak_o.at[pl.ds(r, CHUNK)]))
        jobs.append((hard_hbm.at[pl.ds(r, CHUNK)], hard_o.at[pl.ds(r, CHUNK)]))
    r = wid * LB_PER_W
    jobs.append((lb_hbm.at[pl.ds(r, CHUNK)], lb_o.at[pl.ds(r, CHUNK)]))

    s0, d0 = jobs[0]
    pltpu.async_copy(s0, buf.at[0], sems.at[0]).wait()
    for k in range(len(jobs)):
        b = k % 2
        if k + 1 < len(jobs):
            nb = (k + 1) % 2
            cr = pltpu.async_copy(jobs[k + 1][0], buf.at[nb], sems.at[nb])
        cw = pltpu.async_copy(buf.at[b], jobs[k][1], sems.at[2])
        cw.wait()
        if k + 1 < len(jobs):
            cr.wait()


def _atten_body(w_ref, part_ref, cnt_ref, out_ref):
    psum = part_ref[0, :NUM_CLASSES, :] + part_ref[1, :NUM_CLASSES, :]
    p = psum / cnt_ref[...]
    p = p / jnp.maximum(
        jnp.sqrt(jnp.sum(p * p, axis=1, keepdims=True)), 1e-12)
    w = w_ref[...]
    wn = w / jnp.maximum(
        jnp.sqrt(jnp.sum(w * w, axis=1, keepdims=True)), 1e-12)
    lT = lax.dot_general(
        p, wn, (((1,), (1,)), ((), ())),
        preferred_element_type=jnp.float32) * (1.0 / TAU)
    m = jnp.max(lT, axis=0, keepdims=True)
    e = jnp.exp(lT - m)
    out_ref[...] = e / jnp.sum(e, axis=0, keepdims=True)


def kernel(weak_feat, hard_feat, lb_feat, lb_one_hot, logits_x_lb,
           logits_x_ulb_1, logits_x_ulb_2, y_lb, y_ulb):
    idx_row, class_num = pl.pallas_call(
        _prep_body,
        out_shape=[
            jax.ShapeDtypeStruct((1, B_ULB), jnp.int32),
            jax.ShapeDtypeStruct((NUM_CLASSES, 1), jnp.float32),
        ],
    )(logits_x_ulb_1.T, lb_one_hot.T)

    idx_ulb = idx_row.reshape(B_ULB)
    idx_lb = y_lb.astype(jnp.int32).reshape(B_LB)

    mesh = plsc.VectorSubcoreMesh(
        core_axis_name="c", subcore_axis_name="s",
        num_cores=NC, num_subcores=NS)

    sc_scatter = functools.partial(
        pl.kernel,
        out_type=jax.ShapeDtypeStruct((NC, ACC_ROWS, FEAT_DIM), jnp.float32),
        mesh=mesh,
        scratch_types=[
            pltpu.VMEM((2, CHUNK, FEAT_DIM), jnp.float32),
            pltpu.VMEM((2, CHUNK), jnp.int32),
            pltpu.VMEM((ACC_ROWS, FEAT_DIM), jnp.float32),
            pltpu.SemaphoreType.DMA((2,)),
            pltpu.VMEM_SHARED((ACC_ROWS, FEAT_DIM), jnp.float32),
        ],
    )(_sc_scatter_body)
    partials = sc_scatter(weak_feat, idx_ulb, lb_feat, idx_lb)

    (partials, weak_e, hard_e, lb_e) = lax.optimization_barrier(
        (partials, weak_feat, hard_feat, lb_feat))

    sc_echo = functools.partial(
        pl.kernel,
        out_type=(
            jax.ShapeDtypeStruct((B_ULB, FEAT_DIM), jnp.float32),
            jax.ShapeDtypeStruct((B_ULB, FEAT_DIM), jnp.float32),
            jax.ShapeDtypeStruct((B_LB, FEAT_DIM), jnp.float32),
        ),
        mesh=mesh,
        scratch_types=[
            pltpu.VMEM((2, CHUNK, FEAT_DIM), jnp.float32),
            pltpu.SemaphoreType.DMA((3,)),
        ],
    )(_sc_echo_body)
    weak_o, hard_o, lb_o = sc_echo(weak_e, hard_e, lb_e)

    n_blocks = 16
    blk = B_ULB // n_blocks
    agg_t = pl.pallas_call(
        _atten_body,
        grid=(n_blocks,),
        in_specs=[
            pl.BlockSpec((blk, FEAT_DIM), lambda i: (i, 0)),
            pl.BlockSpec((NC, ACC_ROWS, FEAT_DIM), lambda i: (0, 0, 0)),
            pl.BlockSpec((NUM_CLASSES, 1), lambda i: (0, 0)),
        ],
        out_specs=pl.BlockSpec((NUM_CLASSES, blk), lambda i: (0, i)),
        out_shape=jax.ShapeDtypeStruct((NUM_CLASSES, B_ULB), jnp.float32),
    )(weak_feat, partials, class_num)
    agg_out = agg_t.T

    return (weak_o, hard_o, lb_o, lb_one_hot, logits_x_lb, agg_out,
            logits_x_ulb_2)


# speedup vs baseline: 3.1137x; 1.0143x over previous
import functools

import jax
import jax.numpy as jnp
from jax import lax
from jax.experimental import pallas as pl
from jax.experimental.pallas import tpu as pltpu
from jax.experimental.pallas import tpu_sc as plsc

NUM_CLASSES = 10
FEAT_DIM = 128
TAU = 0.5
P_CUTOFF = 0.5
B_ULB = 16384
B_LB = 4096
NC, NS = 2, 16
NW = NC * NS
ACC_ROWS = 16
CHUNK = 128
ULB_PER_W = B_ULB // NW
LB_PER_W = B_LB // NW
N_CHUNKS = ULB_PER_W // CHUNK + 1


def _prep_body(lT_ref, ohT_ref, idx_ref, cnt_ref):
    lT = lT_ref[...]
    m = jnp.max(lT, axis=0, keepdims=True)
    e = jnp.exp(lT - m)
    s = jnp.sum(e, axis=0, keepdims=True)
    mask = (1.0 / s) > P_CUTOFF
    row = lax.broadcasted_iota(jnp.int32, lT.shape, 0)
    is_max = lT == m
    amax = jnp.min(jnp.where(is_max, row, NUM_CLASSES), axis=0, keepdims=True)
    idx_ref[...] = jnp.where(mask, amax, NUM_CLASSES)
    oh = jnp.where((row == amax) & mask, 1.0, 0.0)
    cnt_ref[...] = (jnp.sum(oh, axis=1, keepdims=True)
                    + jnp.sum(ohT_ref[...], axis=1, keepdims=True))


def _sc_scatter_body(weak_hbm, idxu_hbm, lb_hbm, idxl_hbm, out_hbm,
                     buf, idxv, zbuf, sems, shared):
    cid = lax.axis_index("c")
    sid = lax.axis_index("s")
    wid = cid * NS + sid
    zero = jnp.zeros((16,), jnp.float32)

    @pl.loop(0, ACC_ROWS)
    def _zrow(i):
        for j in range(FEAT_DIM // 16):
            zbuf[i, pl.ds(j * 16, 16)] = zero

    @pl.when(sid == 0)
    def _zero_acc():
        pltpu.sync_copy(zbuf, shared)

    plsc.subcore_barrier()

    @pl.loop(0, ULB_PER_W // CHUNK, step=2)
    def _ulb(k):
        base = wid * ULB_PER_W + k * CHUNK
        ca = pltpu.async_copy(weak_hbm.at[pl.ds(base, CHUNK)],
                              buf.at[0], sems.at[0])
        cia = pltpu.async_copy(idxu_hbm.at[pl.ds(base, CHUNK)],
                               idxv.at[0], sems.at[0])
        cb = pltpu.async_copy(weak_hbm.at[pl.ds(base + CHUNK, CHUNK)],
                              buf.at[1], sems.at[1])
        cib = pltpu.async_copy(idxu_hbm.at[pl.ds(base + CHUNK, CHUNK)],
                               idxv.at[1], sems.at[1])
        ca.wait()
        cia.wait()
        pltpu.sync_copy(buf.at[0], shared.at[idxv.at[0]], add=True)
        cb.wait()
        cib.wait()
        pltpu.sync_copy(buf.at[1], shared.at[idxv.at[1]], add=True)

    base_l = wid * LB_PER_W
    pltpu.sync_copy(idxl_hbm.at[pl.ds(base_l, CHUNK)], idxv.at[0])
    pltpu.sync_copy(lb_hbm.at[pl.ds(base_l, CHUNK)], buf.at[0])
    pltpu.sync_copy(buf.at[0], shared.at[idxv.at[0]], add=True)

    plsc.subcore_barrier()

    @pl.when(sid == 0)
    def _writeback():
        pltpu.sync_copy(shared, zbuf)
        pltpu.sync_copy(zbuf, out_hbm.at[cid])


def _sc_echo_body(weak_hbm, hard_hbm, lb_hbm,
                  weak_o, hard_o, lb_o,
                  buf, sems):
    cid = lax.axis_index("c")
    sid = lax.axis_index("s")
    wid = cid * NS + sid
    E = 2 * CHUNK
    ru = wid * ULB_PER_W
    rl = wid * LB_PER_W
    jobs = [
        (weak_hbm.at[pl.ds(ru, E)], weak_o.at[pl.ds(ru, E)], E),
        (hard_hbm.at[pl.ds(ru, E)], hard_o.at[pl.ds(ru, E)], E),
        (weak_hbm.at[pl.ds(ru + E, E)], weak_o.at[pl.ds(ru + E, E)], E),
        (hard_hbm.at[pl.ds(ru + E, E)], hard_o.at[pl.ds(ru + E, E)], E),
        (lb_hbm.at[pl.ds(rl, CHUNK)], lb_o.at[pl.ds(rl, CHUNK)], CHUNK),
    ]
    n = len(jobs)

    def bslice(b, rows):
        return buf.at[b, pl.ds(0, rows)]

    rs = [pltpu.async_copy(jobs[k][0], bslice(k, jobs[k][2]), sems.at[k])
          for k in range(2)]
    ws = [None] * n
    for k in range(n):
        b = k % 3
        rs[k].wait()
        ws[k] = pltpu.async_copy(bslice(b, jobs[k][2]), jobs[k][1],
                                 sems.at[3 + b])
        if k + 2 < n:
            if k - 1 >= 0:
                ws[k - 1].wait()
            rs.append(pltpu.async_copy(jobs[k + 2][0],
                                       bslice((k + 2) % 3, jobs[k + 2][2]),
                                       sems.at[(k + 2) % 3]))
    ws[n - 3].wait()
    ws[n - 2].wait()
    ws[n - 1].wait()


def _atten_body(w_ref, part_ref, cnt_ref, out_ref, pn_ref):
    @pl.when(pl.program_id(0) == 0)
    def _proto():
        psum = part_ref[0, :NUM_CLASSES, :] + part_ref[1, :NUM_CLASSES, :]
        p = psum / cnt_ref[...]
        pn_ref[...] = p / jnp.maximum(
            jnp.sqrt(jnp.sum(p * p, axis=1, keepdims=True)), 1e-12)

    p = pn_ref[...]
    w = w_ref[...]
    ss = lax.dot_general(
        w * w, jnp.ones((FEAT_DIM, 1), jnp.float32),
        (((1,), (0,)), ((), ())), preferred_element_type=jnp.float32)
    wn = w * (1.0 / jnp.maximum(jnp.sqrt(ss), 1e-12))
    lT = lax.dot_general(
        p, wn, (((1,), (1,)), ((), ())),
        preferred_element_type=jnp.float32) * (1.0 / TAU)
    m = jnp.max(lT, axis=0, keepdims=True)
    e = jnp.exp(lT - m)
    out_ref[...] = e * (1.0 / jnp.sum(e, axis=0, keepdims=True))


def kernel(weak_feat, hard_feat, lb_feat, lb_one_hot, logits_x_lb,
           logits_x_ulb_1, logits_x_ulb_2, y_lb, y_ulb):
    idx_row, class_num = pl.pallas_call(
        _prep_body,
        out_shape=[
            jax.ShapeDtypeStruct((1, B_ULB), jnp.int32),
            jax.ShapeDtypeStruct((NUM_CLASSES, 1), jnp.float32),
        ],
    )(logits_x_ulb_1.T, lb_one_hot.T)

    idx_ulb = idx_row.reshape(B_ULB)
    idx_lb = y_lb.astype(jnp.int32).reshape(B_LB)

    mesh = plsc.VectorSubcoreMesh(
        core_axis_name="c", subcore_axis_name="s",
        num_cores=NC, num_subcores=NS)

    sc_scatter = functools.partial(
        pl.kernel,
        out_type=jax.ShapeDtypeStruct((NC, ACC_ROWS, FEAT_DIM), jnp.float32),
        mesh=mesh,
        scratch_types=[
            pltpu.VMEM((2, CHUNK, FEAT_DIM), jnp.float32),
            pltpu.VMEM((2, CHUNK), jnp.int32),
            pltpu.VMEM((ACC_ROWS, FEAT_DIM), jnp.float32),
            pltpu.SemaphoreType.DMA((2,)),
            pltpu.VMEM_SHARED((ACC_ROWS, FEAT_DIM), jnp.float32),
        ],
    )(_sc_scatter_body)
    partials = sc_scatter(weak_feat, idx_ulb, lb_feat, idx_lb)

    (partials, weak_e, hard_e, lb_e) = lax.optimization_barrier(
        (partials, weak_feat, hard_feat, lb_feat))

    sc_echo = functools.partial(
        pl.kernel,
        out_type=(
            jax.ShapeDtypeStruct((B_ULB, FEAT_DIM), jnp.float32),
            jax.ShapeDtypeStruct((B_ULB, FEAT_DIM), jnp.float32),
            jax.ShapeDtypeStruct((B_LB, FEAT_DIM), jnp.float32),
        ),
        mesh=mesh,
        scratch_types=[
            pltpu.VMEM((3, 2 * CHUNK, FEAT_DIM), jnp.float32),
            pltpu.SemaphoreType.DMA((6,)),
        ],
    )(_sc_echo_body)
    weak_o, hard_o, lb_o = sc_echo(weak_e, hard_e, lb_e)

    n_blocks = 16
    blk = B_ULB // n_blocks
    agg_t = pl.pallas_call(
        _atten_body,
        grid=(n_blocks,),
        in_specs=[
            pl.BlockSpec((blk, FEAT_DIM), lambda i: (i, 0)),
            pl.BlockSpec((NC, ACC_ROWS, FEAT_DIM), lambda i: (0, 0, 0)),
            pl.BlockSpec((NUM_CLASSES, 1), lambda i: (0, 0)),
        ],
        out_specs=pl.BlockSpec((NUM_CLASSES, blk), lambda i: (0, i)),
        out_shape=jax.ShapeDtypeStruct((NUM_CLASSES, B_ULB), jnp.float32),
        scratch_shapes=[pltpu.VMEM((NUM_CLASSES, FEAT_DIM), jnp.float32)],
    )(weak_feat, partials, class_num)
    agg_out = agg_t.T

    return (weak_o, hard_o, lb_o, lb_one_hot, logits_x_lb, agg_out,
            logits_x_ulb_2)


# speedup vs baseline: 3.1985x; 1.0273x over previous
import functools

import jax
import jax.numpy as jnp
from jax import lax
from jax.experimental import pallas as pl
from jax.experimental.pallas import tpu as pltpu
from jax.experimental.pallas import tpu_sc as plsc

NUM_CLASSES = 10
FEAT_DIM = 128
TAU = 0.5
P_CUTOFF = 0.5
B_ULB = 16384
B_LB = 4096
NC, NS = 2, 16
NW = NC * NS
ACC_ROWS = 16
CHUNK = 128
ULB_PER_W = B_ULB // NW
LB_PER_W = B_LB // NW
N_CHUNKS = ULB_PER_W // CHUNK + 1


def _prep_body(lT_ref, ohT_ref, idx_ref, cnt_ref):
    lT = lT_ref[...]
    m = jnp.max(lT, axis=0, keepdims=True)
    e = jnp.exp(lT - m)
    s = jnp.sum(e, axis=0, keepdims=True)
    mask = (1.0 / s) > P_CUTOFF
    row = lax.broadcasted_iota(jnp.int32, lT.shape, 0)
    is_max = lT == m
    amax = jnp.min(jnp.where(is_max, row, NUM_CLASSES), axis=0, keepdims=True)
    idx_ref[...] = jnp.where(mask, amax, NUM_CLASSES)
    oh = jnp.where((row == amax) & mask, 1.0, 0.0)
    cnt_ref[...] = (jnp.sum(oh, axis=1, keepdims=True)
                    + jnp.sum(ohT_ref[...], axis=1, keepdims=True))


def _sc_scatter_body(weak_hbm, idxu_hbm, lb_hbm, idxl_hbm, out_hbm,
                     buf, idxv, zbuf, sems, shared):
    cid = lax.axis_index("c")
    sid = lax.axis_index("s")
    wid = cid * NS + sid
    zero = jnp.zeros((16,), jnp.float32)

    @pl.loop(0, ACC_ROWS)
    def _zrow(i):
        for j in range(FEAT_DIM // 16):
            zbuf[i, pl.ds(j * 16, 16)] = zero

    @pl.when(sid == 0)
    def _zero_acc():
        pltpu.sync_copy(zbuf, shared)

    plsc.subcore_barrier()

    @pl.loop(0, ULB_PER_W // CHUNK, step=2)
    def _ulb(k):
        base = wid * ULB_PER_W + k * CHUNK
        ca = pltpu.async_copy(weak_hbm.at[pl.ds(base, CHUNK)],
                              buf.at[0], sems.at[0])
        cia = pltpu.async_copy(idxu_hbm.at[pl.ds(base, CHUNK)],
                               idxv.at[0], sems.at[0])
        cb = pltpu.async_copy(weak_hbm.at[pl.ds(base + CHUNK, CHUNK)],
                              buf.at[1], sems.at[1])
        cib = pltpu.async_copy(idxu_hbm.at[pl.ds(base + CHUNK, CHUNK)],
                               idxv.at[1], sems.at[1])
        ca.wait()
        cia.wait()
        pltpu.sync_copy(buf.at[0], shared.at[idxv.at[0]], add=True)
        cb.wait()
        cib.wait()
        pltpu.sync_copy(buf.at[1], shared.at[idxv.at[1]], add=True)

    base_l = wid * LB_PER_W
    pltpu.sync_copy(idxl_hbm.at[pl.ds(base_l, CHUNK)], idxv.at[0])
    pltpu.sync_copy(lb_hbm.at[pl.ds(base_l, CHUNK)], buf.at[0])
    pltpu.sync_copy(buf.at[0], shared.at[idxv.at[0]], add=True)

    plsc.subcore_barrier()

    @pl.when(sid == 0)
    def _writeback():
        pltpu.sync_copy(shared, zbuf)
        pltpu.sync_copy(zbuf, out_hbm.at[cid])


def _sc_echo_body(weak_hbm, hard_hbm, lb_hbm,
                  weak_o, hard_o, lb_o,
                  buf, sems):
    cid = lax.axis_index("c")
    sid = lax.axis_index("s")
    wid = cid * NS + sid
    E = 2 * CHUNK
    ru = wid * ULB_PER_W
    rl = wid * LB_PER_W
    jobs = [
        (weak_hbm.at[pl.ds(ru, E)], weak_o.at[pl.ds(ru, E)], E),
        (hard_hbm.at[pl.ds(ru, E)], hard_o.at[pl.ds(ru, E)], E),
        (weak_hbm.at[pl.ds(ru + E, E)], weak_o.at[pl.ds(ru + E, E)], E),
        (hard_hbm.at[pl.ds(ru + E, E)], hard_o.at[pl.ds(ru + E, E)], E),
        (lb_hbm.at[pl.ds(rl, CHUNK)], lb_o.at[pl.ds(rl, CHUNK)], CHUNK),
    ]
    n = len(jobs)

    def bslice(b, rows):
        return buf.at[b, pl.ds(0, rows)]

    rs = [pltpu.async_copy(jobs[k][0], bslice(k, jobs[k][2]), sems.at[k])
          for k in range(2)]
    ws = [None] * n
    for k in range(n):
        b = k % 3
        rs[k].wait()
        ws[k] = pltpu.async_copy(bslice(b, jobs[k][2]), jobs[k][1],
                                 sems.at[3 + b])
        if k + 2 < n:
            if k - 1 >= 0:
                ws[k - 1].wait()
            rs.append(pltpu.async_copy(jobs[k + 2][0],
                                       bslice((k + 2) % 3, jobs[k + 2][2]),
                                       sems.at[(k + 2) % 3]))
    ws[n - 3].wait()
    ws[n - 2].wait()
    ws[n - 1].wait()


def _atten_body(w_ref, part_ref, cnt_ref, out_ref, pn_ref):
    @pl.when(pl.program_id(0) == 0)
    def _proto():
        psum = part_ref[0, :NUM_CLASSES, :] + part_ref[1, :NUM_CLASSES, :]
        p = psum / cnt_ref[...]
        pn_ref[...] = p / jnp.maximum(
            jnp.sqrt(jnp.sum(p * p, axis=1, keepdims=True)), 1e-12)

    p = pn_ref[...]
    w = w_ref[...]
    ssT = lax.dot_general(
        jnp.ones((1, FEAT_DIM), jnp.float32), w * w,
        (((1,), (1,)), ((), ())), preferred_element_type=jnp.float32)
    inv = (1.0 / TAU) / jnp.maximum(jnp.sqrt(ssT), 1e-12)
    lT = lax.dot_general(
        p, w, (((1,), (1,)), ((), ())),
        preferred_element_type=jnp.float32) * inv
    m = jnp.max(lT, axis=0, keepdims=True)
    e = jnp.exp(lT - m)
    out_ref[...] = e * (1.0 / jnp.sum(e, axis=0, keepdims=True))


def kernel(weak_feat, hard_feat, lb_feat, lb_one_hot, logits_x_lb,
           logits_x_ulb_1, logits_x_ulb_2, y_lb, y_ulb):
    idx_row, class_num = pl.pallas_call(
        _prep_body,
        out_shape=[
            jax.ShapeDtypeStruct((1, B_ULB), jnp.int32),
            jax.ShapeDtypeStruct((NUM_CLASSES, 1), jnp.float32),
        ],
    )(logits_x_ulb_1.T, lb_one_hot.T)

    idx_ulb = idx_row.reshape(B_ULB)
    idx_lb = y_lb.astype(jnp.int32).reshape(B_LB)

    mesh = plsc.VectorSubcoreMesh(
        core_axis_name="c", subcore_axis_name="s",
        num_cores=NC, num_subcores=NS)

    sc_scatter = functools.partial(
        pl.kernel,
        out_type=jax.ShapeDtypeStruct((NC, ACC_ROWS, FEAT_DIM), jnp.float32),
        mesh=mesh,
        scratch_types=[
            pltpu.VMEM((2, CHUNK, FEAT_DIM), jnp.float32),
            pltpu.VMEM((2, CHUNK), jnp.int32),
            pltpu.VMEM((ACC_ROWS, FEAT_DIM), jnp.float32),
            pltpu.SemaphoreType.DMA((2,)),
            pltpu.VMEM_SHARED((ACC_ROWS, FEAT_DIM), jnp.float32),
        ],
    )(_sc_scatter_body)
    partials = sc_scatter(weak_feat, idx_ulb, lb_feat, idx_lb)

    (partials, weak_e, hard_e, lb_e) = lax.optimization_barrier(
        (partials, weak_feat, hard_feat, lb_feat))

    sc_echo = functools.partial(
        pl.kernel,
        out_type=(
            jax.ShapeDtypeStruct((B_ULB, FEAT_DIM), jnp.float32),
            jax.ShapeDtypeStruct((B_ULB, FEAT_DIM), jnp.float32),
            jax.ShapeDtypeStruct((B_LB, FEAT_DIM), jnp.float32),
        ),
        mesh=mesh,
        scratch_types=[
            pltpu.VMEM((3, 2 * CHUNK, FEAT_DIM), jnp.float32),
            pltpu.SemaphoreType.DMA((6,)),
        ],
    )(_sc_echo_body)
    weak_o, hard_o, lb_o = sc_echo(weak_e, hard_e, lb_e)

    n_blocks = 16
    blk = B_ULB // n_blocks
    agg_t = pl.pallas_call(
        _atten_body,
        grid=(n_blocks,),
        in_specs=[
            pl.BlockSpec((blk, FEAT_DIM), lambda i: (i, 0)),
            pl.BlockSpec((NC, ACC_ROWS, FEAT_DIM), lambda i: (0, 0, 0)),
            pl.BlockSpec((NUM_CLASSES, 1), lambda i: (0, 0)),
        ],
        out_specs=pl.BlockSpec((NUM_CLASSES, blk), lambda i: (0, i)),
        out_shape=jax.ShapeDtypeStruct((NUM_CLASSES, B_ULB), jnp.float32),
        scratch_shapes=[pltpu.VMEM((NUM_CLASSES, FEAT_DIM), jnp.float32)],
    )(weak_feat, partials, class_num)
    agg_out = agg_t.T

    return (weak_o, hard_o, lb_o, lb_one_hot, logits_x_lb, agg_out,
            logits_x_ulb_2)


# speedup vs baseline: 3.2532x; 1.0171x over previous
import functools

import jax
import jax.numpy as jnp
from jax import lax
from jax.experimental import pallas as pl
from jax.experimental.pallas import tpu as pltpu
from jax.experimental.pallas import tpu_sc as plsc

NUM_CLASSES = 10
FEAT_DIM = 128
TAU = 0.5
P_CUTOFF = 0.5
B_ULB = 16384
B_LB = 4096
NC, NS = 2, 16
NW = NC * NS
ACC_ROWS = 16
CHUNK = 128
ULB_PER_W = B_ULB // NW
LB_PER_W = B_LB // NW
N_CHUNKS = ULB_PER_W // CHUNK + 1


def _prep_body(lT_ref, ohT_ref, idx_ref, cnt_ref):
    lT = lT_ref[...]
    m = jnp.max(lT, axis=0, keepdims=True)
    e = jnp.exp(lT - m)
    s = jnp.sum(e, axis=0, keepdims=True)
    mask = (1.0 / s) > P_CUTOFF
    row = lax.broadcasted_iota(jnp.int32, lT.shape, 0)
    is_max = lT == m
    amax = jnp.min(jnp.where(is_max, row, NUM_CLASSES), axis=0, keepdims=True)
    idx_ref[...] = jnp.where(mask, amax, NUM_CLASSES)
    oh = jnp.where((row == amax) & mask, 1.0, 0.0)
    cnt_ref[...] = (jnp.sum(oh, axis=1, keepdims=True)
                    + jnp.sum(ohT_ref[...], axis=1, keepdims=True))


def _sc_scatter_echo_body(weak_hbm, idxu_hbm, lb_hbm, idxl_hbm, hard_hbm,
                          out_hbm, weak_o, hard_o, lb_o,
                          buf, idxv, zbuf, sems, isems, shared):
    cid = lax.axis_index("c")
    sid = lax.axis_index("s")
    wid = cid * NS + sid
    zero = jnp.zeros((16,), jnp.float32)

    @pl.loop(0, ACC_ROWS)
    def _zrow(i):
        for j in range(FEAT_DIM // 16):
            zbuf[i, pl.ds(j * 16, 16)] = zero

    @pl.when(sid == 0)
    def _zero_acc():
        pltpu.sync_copy(zbuf, shared)

    plsc.subcore_barrier()

    ru = wid * ULB_PER_W
    rl = wid * LB_PER_W
    jobs = []
    for j in range(ULB_PER_W // CHUNK):
        jobs.append((weak_hbm.at[pl.ds(ru + j * CHUNK, CHUNK)],
                     weak_o.at[pl.ds(ru + j * CHUNK, CHUNK)],
                     idxu_hbm.at[pl.ds(ru + j * CHUNK, CHUNK)]))
        jobs.append((hard_hbm.at[pl.ds(ru + j * CHUNK, CHUNK)],
                     hard_o.at[pl.ds(ru + j * CHUNK, CHUNK)], None))
    jobs.append((lb_hbm.at[pl.ds(rl, CHUNK)], lb_o.at[pl.ds(rl, CHUNK)],
                 idxl_hbm.at[pl.ds(rl, CHUNK)]))
    n = len(jobs)

    def issue_read(k):
        r = pltpu.async_copy(jobs[k][0], buf.at[k % 3], sems.at[k % 3])
        ri = None
        if jobs[k][2] is not None:
            ri = pltpu.async_copy(jobs[k][2], idxv.at[k % 2],
                                  isems.at[k % 2])
        return r, ri

    rs = [issue_read(0), issue_read(1)]
    ws = [None] * n
    for k in range(n):
        b = k % 3
        r, ri = rs[k]
        r.wait()
        if ri is not None:
            ri.wait()
            pltpu.sync_copy(buf.at[b], shared.at[idxv.at[k % 2]], add=True)
        ws[k] = pltpu.async_copy(buf.at[b], jobs[k][1], sems.at[3 + b])
        if k + 2 < n:
            if k - 1 >= 0:
                ws[k - 1].wait()
            rs.append(issue_read(k + 2))
    ws[n - 3].wait()
    ws[n - 2].wait()
    ws[n - 1].wait()

    plsc.subcore_barrier()

    @pl.when(sid == 0)
    def _writeback():
        pltpu.sync_copy(shared, zbuf)
        pltpu.sync_copy(zbuf, out_hbm.at[cid])


def _atten_body(w_ref, part_ref, cnt_ref, out_ref, pn_ref):
    @pl.when(pl.program_id(0) == 0)
    def _proto():
        psum = part_ref[0, :NUM_CLASSES, :] + part_ref[1, :NUM_CLASSES, :]
        p = psum / cnt_ref[...]
        pn_ref[...] = p / jnp.maximum(
            jnp.sqrt(jnp.sum(p * p, axis=1, keepdims=True)), 1e-12)

    p = pn_ref[...]
    w = w_ref[...]
    ssT = lax.dot_general(
        jnp.ones((1, FEAT_DIM), jnp.float32), w * w,
        (((1,), (1,)), ((), ())), preferred_element_type=jnp.float32)
    inv = (1.0 / TAU) / jnp.maximum(jnp.sqrt(ssT), 1e-12)
    lT = lax.dot_general(
        p, w, (((1,), (1,)), ((), ())),
        preferred_element_type=jnp.float32) * inv
    m = jnp.max(lT, axis=0, keepdims=True)
    e = jnp.exp(lT - m)
    out_ref[...] = e * (1.0 / jnp.sum(e, axis=0, keepdims=True))


def kernel(weak_feat, hard_feat, lb_feat, lb_one_hot, logits_x_lb,
           logits_x_ulb_1, logits_x_ulb_2, y_lb, y_ulb):
    idx_row, class_num = pl.pallas_call(
        _prep_body,
        out_shape=[
            jax.ShapeDtypeStruct((1, B_ULB), jnp.int32),
            jax.ShapeDtypeStruct((NUM_CLASSES, 1), jnp.float32),
        ],
    )(logits_x_ulb_1.T, lb_one_hot.T)

    idx_ulb = idx_row.reshape(B_ULB)
    idx_lb = y_lb.astype(jnp.int32).reshape(B_LB)

    mesh = plsc.VectorSubcoreMesh(
        core_axis_name="c", subcore_axis_name="s",
        num_cores=NC, num_subcores=NS)

    sc_scatter_echo = functools.partial(
        pl.kernel,
        out_type=(
            jax.ShapeDtypeStruct((NC, ACC_ROWS, FEAT_DIM), jnp.float32),
            jax.ShapeDtypeStruct((B_ULB, FEAT_DIM), jnp.float32),
            jax.ShapeDtypeStruct((B_ULB, FEAT_DIM), jnp.float32),
            jax.ShapeDtypeStruct((B_LB, FEAT_DIM), jnp.float32),
        ),
        mesh=mesh,
        scratch_types=[
            pltpu.VMEM((3, CHUNK, FEAT_DIM), jnp.float32),
            pltpu.VMEM((2, CHUNK), jnp.int32),
            pltpu.VMEM((ACC_ROWS, FEAT_DIM), jnp.float32),
            pltpu.SemaphoreType.DMA((6,)),
            pltpu.SemaphoreType.DMA((2,)),
            pltpu.VMEM_SHARED((ACC_ROWS, FEAT_DIM), jnp.float32),
        ],
    )(_sc_scatter_echo_body)
    partials, weak_o, hard_o, lb_o = sc_scatter_echo(
        weak_feat, idx_ulb, lb_feat, idx_lb, hard_feat)

    n_blocks = 16
    blk = B_ULB // n_blocks
    agg_t = pl.pallas_call(
        _atten_body,
        grid=(n_blocks,),
        in_specs=[
            pl.BlockSpec((blk, FEAT_DIM), lambda i: (i, 0)),
            pl.BlockSpec((NC, ACC_ROWS, FEAT_DIM), lambda i: (0, 0, 0)),
            pl.BlockSpec((NUM_CLASSES, 1), lambda i: (0, 0)),
        ],
        out_specs=pl.BlockSpec((NUM_CLASSES, blk), lambda i: (0, i)),
        out_shape=jax.ShapeDtypeStruct((NUM_CLASSES, B_ULB), jnp.float32),
        scratch_shapes=[pltpu.VMEM((NUM_CLASSES, FEAT_DIM), jnp.float32)],
    )(weak_feat, partials, class_num)
    agg_out = agg_t.T

    return (weak_o, hard_o, lb_o, lb_one_hot, logits_x_lb, agg_out,
            logits_x_ulb_2)


# speedup vs baseline: 3.3406x; 1.0269x over previous
import functools

import jax
import jax.numpy as jnp
from jax import lax
from jax.experimental import pallas as pl
from jax.experimental.pallas import tpu as pltpu
from jax.experimental.pallas import tpu_sc as plsc

NUM_CLASSES = 10
FEAT_DIM = 128
TAU = 0.5
P_CUTOFF = 0.5
B_ULB = 16384
B_LB = 4096
NC, NS = 2, 16
NW = NC * NS
ACC_ROWS = 16
CHUNK = 128
ULB_PER_W = B_ULB // NW
LB_PER_W = B_LB // NW
N_CHUNKS = ULB_PER_W // CHUNK + 1


def _prep_body(lT_ref, ohT_ref, idx_ref, cnt_ref):
    lT = lT_ref[...]
    m = jnp.max(lT, axis=0, keepdims=True)
    e = jnp.exp(lT - m)
    s = jnp.sum(e, axis=0, keepdims=True)
    mask = (1.0 / s) > P_CUTOFF
    row = lax.broadcasted_iota(jnp.int32, lT.shape, 0)
    is_max = lT == m
    amax = jnp.min(jnp.where(is_max, row, NUM_CLASSES), axis=0, keepdims=True)
    idx_ref[...] = jnp.where(mask, amax, NUM_CLASSES)
    oh = jnp.where((row == amax) & mask, 1.0, 0.0)
    cnt_ref[...] = (jnp.sum(oh, axis=1, keepdims=True)
                    + jnp.sum(ohT_ref[...], axis=1, keepdims=True))


def _sc_scatter_echo_body(weak_hbm, idxu_hbm, lb_hbm, idxl_hbm, hard_hbm,
                          out_hbm, weak_o, hard_o, lb_o,
                          wbuf, hbuf, idxv, zbuf, rsems, wsems, isems,
                          shared):
    cid = lax.axis_index("c")
    sid = lax.axis_index("s")
    wid = cid * NS + sid
    zero = jnp.zeros((16,), jnp.float32)

    @pl.loop(0, ACC_ROWS)
    def _zrow(i):
        for j in range(FEAT_DIM // 16):
            zbuf[i, pl.ds(j * 16, 16)] = zero

    @pl.when(sid == 0)
    def _zero_acc():
        pltpu.sync_copy(zbuf, shared)

    plsc.subcore_barrier()

    ru = wid * ULB_PER_W
    rl = wid * LB_PER_W
    NJ = ULB_PER_W // CHUNK

    def wsrc(g):
        return weak_hbm.at[pl.ds(ru + g * CHUNK, CHUNK)]

    def wdst(g):
        return weak_o.at[pl.ds(ru + g * CHUNK, CHUNK)]

    def hsrc(g):
        return hard_hbm.at[pl.ds(ru + g * CHUNK, CHUNK)]

    def hdst(g):
        return hard_o.at[pl.ds(ru + g * CHUNK, CHUNK)]

    def isrc(g):
        return idxu_hbm.at[pl.ds(ru + g * CHUNK, CHUNK)]

    pltpu.async_copy(wsrc(0), wbuf.at[0], rsems.at[0])
    pltpu.async_copy(isrc(0), idxv.at[0], isems.at[0])
    pltpu.async_copy(hsrc(0), hbuf.at[0], rsems.at[2])

    @pl.loop(0, NJ, step=2)
    def _main(g0):
        for b in range(2):
            g = g0 + b
            nb = 1 - b
            pltpu.make_async_copy(wsrc(g), wbuf.at[b], rsems.at[b]).wait()
            pltpu.make_async_copy(isrc(g), idxv.at[b], isems.at[b]).wait()
            pltpu.sync_copy(wbuf.at[b], shared.at[idxv.at[b]], add=True)

            @pl.when(g > 0)
            def _wait_w():
                pltpu.make_async_copy(
                    wbuf.at[nb], wdst(g - 1), wsems.at[nb]).wait()

            @pl.when(g + 1 < NJ)
            def _pf_w():
                pltpu.async_copy(wsrc(g + 1), wbuf.at[nb], rsems.at[nb])
                pltpu.async_copy(isrc(g + 1), idxv.at[nb], isems.at[nb])

            pltpu.async_copy(wbuf.at[b], wdst(g), wsems.at[b])

            pltpu.make_async_copy(hsrc(g), hbuf.at[b], rsems.at[2 + b]).wait()

            @pl.when(g > 0)
            def _wait_h():
                pltpu.make_async_copy(
                    hbuf.at[nb], hdst(g - 1), wsems.at[2 + nb]).wait()

            @pl.when(g + 1 < NJ)
            def _pf_h():
                pltpu.async_copy(hsrc(g + 1), hbuf.at[nb], rsems.at[2 + nb])

            pltpu.async_copy(hbuf.at[b], hdst(g), wsems.at[2 + b])

    pltpu.make_async_copy(wbuf.at[1], wdst(NJ - 1), wsems.at[1]).wait()
    pltpu.make_async_copy(hbuf.at[1], hdst(NJ - 1), wsems.at[3]).wait()

    pltpu.sync_copy(idxl_hbm.at[pl.ds(rl, CHUNK)], idxv.at[0])
    pltpu.sync_copy(lb_hbm.at[pl.ds(rl, CHUNK)], wbuf.at[0])
    pltpu.sync_copy(wbuf.at[0], shared.at[idxv.at[0]], add=True)
    pltpu.sync_copy(wbuf.at[0], lb_o.at[pl.ds(rl, CHUNK)])

    plsc.subcore_barrier()

    @pl.when(sid == 0)
    def _writeback():
        pltpu.sync_copy(shared, zbuf)
        pltpu.sync_copy(zbuf, out_hbm.at[cid])


def _atten_body(w_ref, part_ref, cnt_ref, out_ref, pn_ref):
    @pl.when(pl.program_id(0) == 0)
    def _proto():
        psum = part_ref[0, :NUM_CLASSES, :] + part_ref[1, :NUM_CLASSES, :]
        p = psum / cnt_ref[...]
        pn_ref[...] = p / jnp.maximum(
            jnp.sqrt(jnp.sum(p * p, axis=1, keepdims=True)), 1e-12)

    p = pn_ref[...]
    w = w_ref[...]
    ssT = lax.dot_general(
        jnp.ones((1, FEAT_DIM), jnp.float32), w * w,
        (((1,), (1,)), ((), ())), preferred_element_type=jnp.float32)
    inv = (1.0 / TAU) / jnp.maximum(jnp.sqrt(ssT), 1e-12)
    lT = lax.dot_general(
        p, w, (((1,), (1,)), ((), ())),
        preferred_element_type=jnp.float32) * inv
    m = jnp.max(lT, axis=0, keepdims=True)
    e = jnp.exp(lT - m)
    out_ref[...] = e * (1.0 / jnp.sum(e, axis=0, keepdims=True))


def kernel(weak_feat, hard_feat, lb_feat, lb_one_hot, logits_x_lb,
           logits_x_ulb_1, logits_x_ulb_2, y_lb, y_ulb):
    idx_row, class_num = pl.pallas_call(
        _prep_body,
        out_shape=[
            jax.ShapeDtypeStruct((1, B_ULB), jnp.int32),
            jax.ShapeDtypeStruct((NUM_CLASSES, 1), jnp.float32),
        ],
    )(logits_x_ulb_1.T, lb_one_hot.T)

    idx_ulb = idx_row.reshape(B_ULB)
    idx_lb = y_lb.astype(jnp.int32).reshape(B_LB)

    mesh = plsc.VectorSubcoreMesh(
        core_axis_name="c", subcore_axis_name="s",
        num_cores=NC, num_subcores=NS)

    sc_scatter_echo = functools.partial(
        pl.kernel,
        out_type=(
            jax.ShapeDtypeStruct((NC, ACC_ROWS, FEAT_DIM), jnp.float32),
            jax.ShapeDtypeStruct((B_ULB, FEAT_DIM), jnp.float32),
            jax.ShapeDtypeStruct((B_ULB, FEAT_DIM), jnp.float32),
            jax.ShapeDtypeStruct((B_LB, FEAT_DIM), jnp.float32),
        ),
        mesh=mesh,
        scratch_types=[
            pltpu.VMEM((2, CHUNK, FEAT_DIM), jnp.float32),
            pltpu.VMEM((2, CHUNK, FEAT_DIM), jnp.float32),
            pltpu.VMEM((2, CHUNK), jnp.int32),
            pltpu.VMEM((ACC_ROWS, FEAT_DIM), jnp.float32),
            pltpu.SemaphoreType.DMA((4,)),
            pltpu.SemaphoreType.DMA((4,)),
            pltpu.SemaphoreType.DMA((2,)),
            pltpu.VMEM_SHARED((ACC_ROWS, FEAT_DIM), jnp.float32),
        ],
    )(_sc_scatter_echo_body)
    partials, weak_o, hard_o, lb_o = sc_scatter_echo(
        weak_feat, idx_ulb, lb_feat, idx_lb, hard_feat)

    n_blocks = 8
    blk = B_ULB // n_blocks
    agg_t = pl.pallas_call(
        _atten_body,
        grid=(n_blocks,),
        in_specs=[
            pl.BlockSpec((blk, FEAT_DIM), lambda i: (i, 0)),
            pl.BlockSpec((NC, ACC_ROWS, FEAT_DIM), lambda i: (0, 0, 0)),
            pl.BlockSpec((NUM_CLASSES, 1), lambda i: (0, 0)),
        ],
        out_specs=pl.BlockSpec((NUM_CLASSES, blk), lambda i: (0, i)),
        out_shape=jax.ShapeDtypeStruct((NUM_CLASSES, B_ULB), jnp.float32),
        scratch_shapes=[pltpu.VMEM((NUM_CLASSES, FEAT_DIM), jnp.float32)],
    )(weak_feat, partials, class_num)
    agg_out = agg_t.T

    return (weak_o, hard_o, lb_o, lb_one_hot, logits_x_lb, agg_out,
            logits_x_ulb_2)


# speedup vs baseline: 3.5065x; 1.0496x over previous
import functools

import jax
import jax.numpy as jnp
from jax import lax
from jax.experimental import pallas as pl
from jax.experimental.pallas import tpu as pltpu
from jax.experimental.pallas import tpu_sc as plsc

NUM_CLASSES = 10
FEAT_DIM = 128
TAU = 0.5
P_CUTOFF = 0.5
B_ULB = 16384
B_LB = 4096
NC, NS = 2, 16
NW = NC * NS
ACC_ROWS = 16
CHUNK = 128
ULB_PER_W = B_ULB // NW
LB_PER_W = B_LB // NW
N_CHUNKS = ULB_PER_W // CHUNK + 1


def _prep_body(lT_ref, ohT_ref, idx_ref, cnt_ref):
    lT = lT_ref[...]
    m = jnp.max(lT, axis=0, keepdims=True)
    e = jnp.exp(lT - m)
    s = jnp.sum(e, axis=0, keepdims=True)
    mask = (1.0 / s) > P_CUTOFF
    row = lax.broadcasted_iota(jnp.int32, lT.shape, 0)
    is_max = lT == m
    amax = jnp.min(jnp.where(is_max, row, NUM_CLASSES), axis=0, keepdims=True)
    idx_ref[...] = jnp.where(mask, amax, NUM_CLASSES)
    oh = jnp.where((row == amax) & mask, 1.0, 0.0)
    cnt_ref[...] = (jnp.sum(oh, axis=1, keepdims=True)
                    + jnp.sum(ohT_ref[...], axis=1, keepdims=True))


def _sc_scatter_echo_body(weak_hbm, idxu_hbm, lb_hbm, idxl_hbm, hard_hbm,
                          out_hbm, weak_o, hard_o, lb_o,
                          buf, idxv, zbuf, rsems, wsems, isems, shared):
    cid = lax.axis_index("c")
    sid = lax.axis_index("s")
    wid = cid * NS + sid
    zero = jnp.zeros((16,), jnp.float32)

    @pl.loop(0, ACC_ROWS)
    def _zrow(i):
        for j in range(FEAT_DIM // 16):
            zbuf[i, pl.ds(j * 16, 16)] = zero

    @pl.when(sid == 0)
    def _zero_acc():
        pltpu.sync_copy(zbuf, shared)

    plsc.subcore_barrier()

    ru = wid * ULB_PER_W
    rl = wid * LB_PER_W
    jobs = []
    for j in range(ULB_PER_W // CHUNK):
        jobs.append((weak_hbm.at[pl.ds(ru + j * CHUNK, CHUNK)],
                     weak_o.at[pl.ds(ru + j * CHUNK, CHUNK)],
                     idxu_hbm.at[pl.ds(ru + j * CHUNK, CHUNK)]))
        jobs.append((hard_hbm.at[pl.ds(ru + j * CHUNK, CHUNK)],
                     hard_o.at[pl.ds(ru + j * CHUNK, CHUNK)], None))
    jobs.append((lb_hbm.at[pl.ds(rl, CHUNK)], lb_o.at[pl.ds(rl, CHUNK)],
                 idxl_hbm.at[pl.ds(rl, CHUNK)]))
    n = len(jobs)

    def issue_read(k):
        r = pltpu.async_copy(jobs[k][0], buf.at[k % 4], rsems.at[k % 4])
        ri = None
        if jobs[k][2] is not None:
            ri = pltpu.async_copy(jobs[k][2], idxv.at[k % 2],
                                  isems.at[k % 2])
        return r, ri

    rs = [issue_read(0), issue_read(1)]
    ws = [None] * n
    for k in range(n):
        b = k % 4
        r, ri = rs[k]
        r.wait()
        if ri is not None:
            ri.wait()
            pltpu.sync_copy(buf.at[b], shared.at[idxv.at[k % 2]], add=True)
        ws[k] = pltpu.async_copy(buf.at[b], jobs[k][1], wsems.at[b])
        if k + 2 < n:
            if k - 2 >= 0:
                ws[k - 2].wait()
            rs.append(issue_read(k + 2))
    ws[n - 4].wait()
    ws[n - 3].wait()
    ws[n - 2].wait()
    ws[n - 1].wait()

    plsc.subcore_barrier()

    @pl.when(sid == 0)
    def _writeback():
        pltpu.sync_copy(shared, zbuf)
        pltpu.sync_copy(zbuf, out_hbm.at[cid])


def _atten_body(w_ref, part_ref, cnt_ref, out_ref, pn_ref):
    @pl.when(pl.program_id(0) == 0)
    def _proto():
        psum = part_ref[0, :NUM_CLASSES, :] + part_ref[1, :NUM_CLASSES, :]
        p = psum / cnt_ref[...]
        pn_ref[...] = p / jnp.maximum(
            jnp.sqrt(jnp.sum(p * p, axis=1, keepdims=True)), 1e-12)

    p = pn_ref[...]
    w = w_ref[...]
    ssT = lax.dot_general(
        jnp.ones((1, FEAT_DIM), jnp.float32), w * w,
        (((1,), (1,)), ((), ())), preferred_element_type=jnp.float32)
    inv = (1.0 / TAU) / jnp.maximum(jnp.sqrt(ssT), 1e-12)
    lT = lax.dot_general(
        p, w, (((1,), (1,)), ((), ())),
        preferred_element_type=jnp.float32) * inv
    m = jnp.max(lT, axis=0, keepdims=True)
    e = jnp.exp(lT - m)
    out_ref[...] = e * (1.0 / jnp.sum(e, axis=0, keepdims=True))


def kernel(weak_feat, hard_feat, lb_feat, lb_one_hot, logits_x_lb,
           logits_x_ulb_1, logits_x_ulb_2, y_lb, y_ulb):
    idx_row, class_num = pl.pallas_call(
        _prep_body,
        out_shape=[
            jax.ShapeDtypeStruct((1, B_ULB), jnp.int32),
            jax.ShapeDtypeStruct((NUM_CLASSES, 1), jnp.float32),
        ],
    )(logits_x_ulb_1.T, lb_one_hot.T)

    idx_ulb = idx_row.reshape(B_ULB)
    idx_lb = y_lb.astype(jnp.int32).reshape(B_LB)

    mesh = plsc.VectorSubcoreMesh(
        core_axis_name="c", subcore_axis_name="s",
        num_cores=NC, num_subcores=NS)

    sc_scatter_echo = functools.partial(
        pl.kernel,
        out_type=(
            jax.ShapeDtypeStruct((NC, ACC_ROWS, FEAT_DIM), jnp.float32),
            jax.ShapeDtypeStruct((B_ULB, FEAT_DIM), jnp.float32),
            jax.ShapeDtypeStruct((B_ULB, FEAT_DIM), jnp.float32),
            jax.ShapeDtypeStruct((B_LB, FEAT_DIM), jnp.float32),
        ),
        mesh=mesh,
        scratch_types=[
            pltpu.VMEM((4, CHUNK, FEAT_DIM), jnp.float32),
            pltpu.VMEM((2, CHUNK), jnp.int32),
            pltpu.VMEM((ACC_ROWS, FEAT_DIM), jnp.float32),
            pltpu.SemaphoreType.DMA((4,)),
            pltpu.SemaphoreType.DMA((4,)),
            pltpu.SemaphoreType.DMA((2,)),
            pltpu.VMEM_SHARED((ACC_ROWS, FEAT_DIM), jnp.float32),
        ],
    )(_sc_scatter_echo_body)
    partials, weak_o, hard_o, lb_o = sc_scatter_echo(
        weak_feat, idx_ulb, lb_feat, idx_lb, hard_feat)

    n_blocks = 8
    blk = B_ULB // n_blocks
    agg_t = pl.pallas_call(
        _atten_body,
        grid=(n_blocks,),
        in_specs=[
            pl.BlockSpec((blk, FEAT_DIM), lambda i: (i, 0)),
            pl.BlockSpec((NC, ACC_ROWS, FEAT_DIM), lambda i: (0, 0, 0)),
            pl.BlockSpec((NUM_CLASSES, 1), lambda i: (0, 0)),
        ],
        out_specs=pl.BlockSpec((NUM_CLASSES, blk), lambda i: (0, i)),
        out_shape=jax.ShapeDtypeStruct((NUM_CLASSES, B_ULB), jnp.float32),
        scratch_shapes=[pltpu.VMEM((NUM_CLASSES, FEAT_DIM), jnp.float32)],
    )(weak_feat, partials, class_num)
    agg_out = agg_t.T

    return (weak_o, hard_o, lb_o, lb_one_hot, logits_x_lb, agg_out,
            logits_x_ulb_2)


# speedup vs baseline: 3.8001x; 1.0838x over previous
import functools

import jax
import jax.numpy as jnp
from jax import lax
from jax.experimental import pallas as pl
from jax.experimental.pallas import tpu as pltpu
from jax.experimental.pallas import tpu_sc as plsc

NUM_CLASSES = 10
FEAT_DIM = 128
TAU = 0.5
P_CUTOFF = 0.5
B_ULB = 16384
B_LB = 4096
NC, NS = 2, 16
NW = NC * NS
ACC_ROWS = 16
CHUNK = 128
ULB_PER_W = B_ULB // NW
LB_PER_W = B_LB // NW
N_CHUNKS = ULB_PER_W // CHUNK + 1


def _prep_body(lT_ref, ohT_ref, llbT_ref, l2T_ref,
               idx_ref, cnt_ref, ohT_o, llbT_o, l2T_o):
    lT = lT_ref[...]
    m = jnp.max(lT, axis=0, keepdims=True)
    e = jnp.exp(lT - m)
    s = jnp.sum(e, axis=0, keepdims=True)
    mask = (1.0 / s) > P_CUTOFF
    row = lax.broadcasted_iota(jnp.int32, lT.shape, 0)
    is_max = lT == m
    amax = jnp.min(jnp.where(is_max, row, NUM_CLASSES), axis=0, keepdims=True)
    idx_ref[...] = jnp.where(mask, amax, NUM_CLASSES)
    oh = jnp.where((row == amax) & mask, 1.0, 0.0)
    ohT = ohT_ref[...]
    cnt_ref[...] = (jnp.sum(oh, axis=1, keepdims=True)
                    + jnp.sum(ohT, axis=1, keepdims=True))
    ohT_o[...] = ohT
    llbT_o[...] = llbT_ref[...]
    l2T_o[...] = l2T_ref[...]


def _sc_scatter_echo_body(weak_hbm, idxu_hbm, lb_hbm, idxl_hbm, hard_hbm,
                          out_hbm, weak_o, hard_o, lb_o,
                          buf, idxv, zbuf, rsems, wsems, isems, shared):
    cid = lax.axis_index("c")
    sid = lax.axis_index("s")
    wid = cid * NS + sid
    zero = jnp.zeros((16,), jnp.float32)

    @pl.loop(0, ACC_ROWS)
    def _zrow(i):
        for j in range(FEAT_DIM // 16):
            zbuf[i, pl.ds(j * 16, 16)] = zero

    @pl.when(sid == 0)
    def _zero_acc():
        pltpu.sync_copy(zbuf, shared)

    plsc.subcore_barrier()

    ru = wid * ULB_PER_W
    rl = wid * LB_PER_W
    jobs = []
    for j in range(ULB_PER_W // CHUNK):
        jobs.append((weak_hbm.at[pl.ds(ru + j * CHUNK, CHUNK)],
                     weak_o.at[pl.ds(ru + j * CHUNK, CHUNK)],
                     idxu_hbm.at[pl.ds(ru + j * CHUNK, CHUNK)]))
        jobs.append((hard_hbm.at[pl.ds(ru + j * CHUNK, CHUNK)],
                     hard_o.at[pl.ds(ru + j * CHUNK, CHUNK)], None))
    jobs.append((lb_hbm.at[pl.ds(rl, CHUNK)], lb_o.at[pl.ds(rl, CHUNK)],
                 idxl_hbm.at[pl.ds(rl, CHUNK)]))
    n = len(jobs)

    def issue_read(k):
        r = pltpu.async_copy(jobs[k][0], buf.at[k % 4], rsems.at[k % 4])
        ri = None
        if jobs[k][2] is not None:
            ri = pltpu.async_copy(jobs[k][2], idxv.at[k % 2],
                                  isems.at[k % 2])
        return r, ri

    rs = [issue_read(0), issue_read(1)]
    ws = [None] * n
    for k in range(n):
        b = k % 4
        r, ri = rs[k]
        r.wait()
        if ri is not None:
            ri.wait()
            pltpu.sync_copy(buf.at[b], shared.at[idxv.at[k % 2]], add=True)
        ws[k] = pltpu.async_copy(buf.at[b], jobs[k][1], wsems.at[b])
        if k + 2 < n:
            if k - 2 >= 0:
                ws[k - 2].wait()
            rs.append(issue_read(k + 2))
    ws[n - 4].wait()
    ws[n - 3].wait()
    ws[n - 2].wait()
    ws[n - 1].wait()

    plsc.subcore_barrier()

    @pl.when(sid == 0)
    def _writeback():
        pltpu.sync_copy(shared, zbuf)
        pltpu.sync_copy(zbuf, out_hbm.at[cid])


def _atten_body(w_ref, part_ref, cnt_ref, out_ref, pn_ref):
    @pl.when(pl.program_id(0) == 0)
    def _proto():
        psum = part_ref[0, :NUM_CLASSES, :] + part_ref[1, :NUM_CLASSES, :]
        p = psum / cnt_ref[...]
        pn_ref[...] = p / jnp.maximum(
            jnp.sqrt(jnp.sum(p * p, axis=1, keepdims=True)), 1e-12)

    p = pn_ref[...]
    w = w_ref[...]
    ssT = lax.dot_general(
        jnp.ones((1, FEAT_DIM), jnp.float32), w * w,
        (((1,), (1,)), ((), ())), preferred_element_type=jnp.float32)
    inv = (1.0 / TAU) / jnp.maximum(jnp.sqrt(ssT), 1e-12)
    lT = lax.dot_general(
        p, w, (((1,), (1,)), ((), ())),
        preferred_element_type=jnp.float32) * inv
    m = jnp.max(lT, axis=0, keepdims=True)
    e = jnp.exp(lT - m)
    out_ref[...] = e * (1.0 / jnp.sum(e, axis=0, keepdims=True))


def kernel(weak_feat, hard_feat, lb_feat, lb_one_hot, logits_x_lb,
           logits_x_ulb_1, logits_x_ulb_2, y_lb, y_ulb):
    idx_row, class_num, ohT_o, llbT_o, l2T_o = pl.pallas_call(
        _prep_body,
        out_shape=[
            jax.ShapeDtypeStruct((1, B_ULB), jnp.int32),
            jax.ShapeDtypeStruct((NUM_CLASSES, 1), jnp.float32),
            jax.ShapeDtypeStruct((NUM_CLASSES, B_LB), jnp.float32),
            jax.ShapeDtypeStruct((NUM_CLASSES, B_LB), jnp.float32),
            jax.ShapeDtypeStruct((NUM_CLASSES, B_ULB), jnp.float32),
        ],
    )(logits_x_ulb_1.T, lb_one_hot.T, logits_x_lb.T, logits_x_ulb_2.T)

    idx_ulb = idx_row.reshape(B_ULB)
    idx_lb = y_lb.astype(jnp.int32).reshape(B_LB)

    mesh = plsc.VectorSubcoreMesh(
        core_axis_name="c", subcore_axis_name="s",
        num_cores=NC, num_subcores=NS)

    sc_scatter_echo = functools.partial(
        pl.kernel,
        out_type=(
            jax.ShapeDtypeStruct((NC, ACC_ROWS, FEAT_DIM), jnp.float32),
            jax.ShapeDtypeStruct((B_ULB, FEAT_DIM), jnp.float32),
            jax.ShapeDtypeStruct((B_ULB, FEAT_DIM), jnp.float32),
            jax.ShapeDtypeStruct((B_LB, FEAT_DIM), jnp.float32),
        ),
        mesh=mesh,
        scratch_types=[
            pltpu.VMEM((4, CHUNK, FEAT_DIM), jnp.float32),
            pltpu.VMEM((2, CHUNK), jnp.int32),
            pltpu.VMEM((ACC_ROWS, FEAT_DIM), jnp.float32),
            pltpu.SemaphoreType.DMA((4,)),
            pltpu.SemaphoreType.DMA((4,)),
            pltpu.SemaphoreType.DMA((2,)),
            pltpu.VMEM_SHARED((ACC_ROWS, FEAT_DIM), jnp.float32),
        ],
    )(_sc_scatter_echo_body)
    partials, weak_o, hard_o, lb_o = sc_scatter_echo(
        weak_feat, idx_ulb, lb_feat, idx_lb, hard_feat)

    n_blocks = 4
    blk = B_ULB // n_blocks
    agg_t = pl.pallas_call(
        _atten_body,
        grid=(n_blocks,),
        in_specs=[
            pl.BlockSpec((blk, FEAT_DIM), lambda i: (i, 0)),
            pl.BlockSpec((NC, ACC_ROWS, FEAT_DIM), lambda i: (0, 0, 0)),
            pl.BlockSpec((NUM_CLASSES, 1), lambda i: (0, 0)),
        ],
        out_specs=pl.BlockSpec((NUM_CLASSES, blk), lambda i: (0, i)),
        out_shape=jax.ShapeDtypeStruct((NUM_CLASSES, B_ULB), jnp.float32),
        scratch_shapes=[pltpu.VMEM((NUM_CLASSES, FEAT_DIM), jnp.float32)],
    )(weak_feat, partials, class_num)
    agg_out = agg_t.T

    return (weak_o, hard_o, lb_o, ohT_o.T, llbT_o.T, agg_out, l2T_o.T)


# speedup vs baseline: 3.9704x; 1.0448x over previous
import functools

import jax
import jax.numpy as jnp
from jax import lax
from jax.experimental import pallas as pl
from jax.experimental.pallas import tpu as pltpu
from jax.experimental.pallas import tpu_sc as plsc

NUM_CLASSES = 10
FEAT_DIM = 128
TAU = 0.5
P_CUTOFF = 0.5
B_ULB = 16384
B_LB = 4096
NC, NS = 2, 16
NW = NC * NS
ACC_ROWS = 16
CHUNK = 128
ULB_PER_W = B_ULB // NW
LB_PER_W = B_LB // NW
N_CHUNKS = ULB_PER_W // CHUNK + 1


def _prep_body(lT_ref, ohT_ref, llbT_ref, l2T_ref,
               idx_ref, cnt_ref, ohT_o, llbT_o, l2T_o):
    lT = lT_ref[...]
    m = jnp.max(lT, axis=0, keepdims=True)
    e = jnp.exp(lT - m)
    s = jnp.sum(e, axis=0, keepdims=True)
    mask = (1.0 / s) > P_CUTOFF
    row = lax.broadcasted_iota(jnp.int32, lT.shape, 0)
    is_max = lT == m
    amax = jnp.min(jnp.where(is_max, row, NUM_CLASSES), axis=0, keepdims=True)
    idx_ref[...] = jnp.where(mask, amax, NUM_CLASSES)
    oh = jnp.where((row == amax) & mask, 1.0, 0.0)
    ohT = ohT_ref[...]
    cnt_ref[...] = (jnp.sum(oh, axis=1, keepdims=True)
                    + jnp.sum(ohT, axis=1, keepdims=True))
    ohT_o[...] = ohT
    llbT_o[...] = llbT_ref[...]
    l2T_o[...] = l2T_ref[...]


def _sc_scatter_echo_body(weak_hbm, idxu_hbm, lb_hbm, idxl_hbm, hard_hbm,
                          out_hbm, weak_o, hard_o, lb_o,
                          buf, idxv, zbuf, rsems, wsems, isems, ssems, shared):
    cid = lax.axis_index("c")
    sid = lax.axis_index("s")
    wid = cid * NS + sid
    zero = jnp.zeros((16,), jnp.float32)

    @pl.loop(0, ACC_ROWS)
    def _zrow(i):
        for j in range(FEAT_DIM // 16):
            zbuf[i, pl.ds(j * 16, 16)] = zero

    @pl.when(sid == 0)
    def _zero_acc():
        pltpu.sync_copy(zbuf, shared)

    plsc.subcore_barrier()

    ru = wid * ULB_PER_W
    rl = wid * LB_PER_W
    jobs = []
    for j in range(ULB_PER_W // CHUNK):
        jobs.append((weak_hbm.at[pl.ds(ru + j * CHUNK, CHUNK)],
                     weak_o.at[pl.ds(ru + j * CHUNK, CHUNK)],
                     idxu_hbm.at[pl.ds(ru + j * CHUNK, CHUNK)]))
        jobs.append((hard_hbm.at[pl.ds(ru + j * CHUNK, CHUNK)],
                     hard_o.at[pl.ds(ru + j * CHUNK, CHUNK)], None))
    jobs.append((lb_hbm.at[pl.ds(rl, CHUNK)], lb_o.at[pl.ds(rl, CHUNK)],
                 idxl_hbm.at[pl.ds(rl, CHUNK)]))
    n = len(jobs)

    def issue_read(k):
        r = pltpu.async_copy(jobs[k][0], buf.at[k % 4], rsems.at[k % 4])
        ri = None
        if jobs[k][2] is not None:
            o = k // 2
            ri = pltpu.async_copy(jobs[k][2], idxv.at[o % 2],
                                  isems.at[o % 2])
        return r, ri

    rs = [issue_read(0), issue_read(1)]
    ws = [None] * n
    ss = []
    for k in range(n):
        b = k % 4
        r, ri = rs[k]
        r.wait()
        if ri is not None:
            ri.wait()
            o = k // 2
            ss.append(pltpu.async_copy(
                buf.at[b], shared.at[idxv.at[o % 2]], ssems.at[o % 2],
                add=True))
        ws[k] = pltpu.async_copy(buf.at[b], jobs[k][1], wsems.at[b])
        if k + 2 < n:
            if k - 2 >= 0:
                ws[k - 2].wait()
                if jobs[k - 2][2] is not None:
                    ss[(k - 2) // 2].wait()
            rs.append(issue_read(k + 2))
    ws[n - 4].wait()
    ws[n - 3].wait()
    ws[n - 2].wait()
    ws[n - 1].wait()
    ss[-1].wait()
    ss[-2].wait()

    plsc.subcore_barrier()

    @pl.when(sid == 0)
    def _writeback():
        pltpu.sync_copy(shared, zbuf)
        pltpu.sync_copy(zbuf, out_hbm.at[cid])


def _atten_body(w_ref, part_ref, cnt_ref, out_ref, pn_ref):
    @pl.when(pl.program_id(0) == 0)
    def _proto():
        psum = part_ref[0, :NUM_CLASSES, :] + part_ref[1, :NUM_CLASSES, :]
        p = psum / cnt_ref[...]
        pn_ref[...] = p / jnp.maximum(
            jnp.sqrt(jnp.sum(p * p, axis=1, keepdims=True)), 1e-12)

    p = pn_ref[...]
    w = w_ref[...]
    ssT = lax.dot_general(
        jnp.ones((1, FEAT_DIM), jnp.float32), w * w,
        (((1,), (1,)), ((), ())), preferred_element_type=jnp.float32)
    inv = (1.0 / TAU) / jnp.maximum(jnp.sqrt(ssT), 1e-12)
    lT = lax.dot_general(
        p, w, (((1,), (1,)), ((), ())),
        preferred_element_type=jnp.float32) * inv
    m = jnp.max(lT, axis=0, keepdims=True)
    e = jnp.exp(lT - m)
    out_ref[...] = e * (1.0 / jnp.sum(e, axis=0, keepdims=True))


def kernel(weak_feat, hard_feat, lb_feat, lb_one_hot, logits_x_lb,
           logits_x_ulb_1, logits_x_ulb_2, y_lb, y_ulb):
    idx_row, class_num, ohT_o, llbT_o, l2T_o = pl.pallas_call(
        _prep_body,
        out_shape=[
            jax.ShapeDtypeStruct((1, B_ULB), jnp.int32),
            jax.ShapeDtypeStruct((NUM_CLASSES, 1), jnp.float32),
            jax.ShapeDtypeStruct((NUM_CLASSES, B_LB), jnp.float32),
            jax.ShapeDtypeStruct((NUM_CLASSES, B_LB), jnp.float32),
            jax.ShapeDtypeStruct((NUM_CLASSES, B_ULB), jnp.float32),
        ],
    )(logits_x_ulb_1.T, lb_one_hot.T, logits_x_lb.T, logits_x_ulb_2.T)

    idx_ulb = idx_row.reshape(B_ULB)
    idx_lb = y_lb.astype(jnp.int32).reshape(B_LB)

    mesh = plsc.VectorSubcoreMesh(
        core_axis_name="c", subcore_axis_name="s",
        num_cores=NC, num_subcores=NS)

    sc_scatter_echo = functools.partial(
        pl.kernel,
        out_type=(
            jax.ShapeDtypeStruct((NC, ACC_ROWS, FEAT_DIM), jnp.float32),
            jax.ShapeDtypeStruct((B_ULB, FEAT_DIM), jnp.float32),
            jax.ShapeDtypeStruct((B_ULB, FEAT_DIM), jnp.float32),
            jax.ShapeDtypeStruct((B_LB, FEAT_DIM), jnp.float32),
        ),
        mesh=mesh,
        scratch_types=[
            pltpu.VMEM((4, CHUNK, FEAT_DIM), jnp.float32),
            pltpu.VMEM((2, CHUNK), jnp.int32),
            pltpu.VMEM((ACC_ROWS, FEAT_DIM), jnp.float32),
            pltpu.SemaphoreType.DMA((4,)),
            pltpu.SemaphoreType.DMA((4,)),
            pltpu.SemaphoreType.DMA((2,)),
            pltpu.SemaphoreType.DMA((2,)),
            pltpu.VMEM_SHARED((ACC_ROWS, FEAT_DIM), jnp.float32),
        ],
    )(_sc_scatter_echo_body)
    partials, weak_o, hard_o, lb_o = sc_scatter_echo(
        weak_feat, idx_ulb, lb_feat, idx_lb, hard_feat)

    n_blocks = 4
    blk = B_ULB // n_blocks
    agg_t = pl.pallas_call(
        _atten_body,
        grid=(n_blocks,),
        in_specs=[
            pl.BlockSpec((blk, FEAT_DIM), lambda i: (i, 0)),
            pl.BlockSpec((NC, ACC_ROWS, FEAT_DIM), lambda i: (0, 0, 0)),
            pl.BlockSpec((NUM_CLASSES, 1), lambda i: (0, 0)),
        ],
        out_specs=pl.BlockSpec((NUM_CLASSES, blk), lambda i: (0, i)),
        out_shape=jax.ShapeDtypeStruct((NUM_CLASSES, B_ULB), jnp.float32),
        scratch_shapes=[pltpu.VMEM((NUM_CLASSES, FEAT_DIM), jnp.float32)],
    )(weak_feat, partials, class_num)
    agg_out = agg_t.T

    return (weak_o, hard_o, lb_o, ohT_o.T, llbT_o.T, agg_out, l2T_o.T)
